# baseline (speedup 1.0000x reference)
import functools

import jax
import jax.numpy as jnp
from jax import lax
from jax.experimental import pallas as pl
from jax.experimental.pallas import tpu as pltpu
from jax.experimental.pallas import tpu_sc as plsc

ACT_VOCAB = 1000
D = 64
CONTEXT = 20
B = 4096

NC = 2
NS = 16
NW = NC * NS
BPW = B // NW
IDX_W = 128
ACT_DMAS = BPW * CONTEXT // IDX_W
HALF = ACT_DMAS // 2
HALF_ROWS = HALF * IDX_W


def _sc_act_gather(act_idx, act_table):
  mesh = plsc.VectorSubcoreMesh(core_axis_name="c", subcore_axis_name="s")

  @functools.partial(
      pl.kernel,
      mesh=mesh,
      compiler_params=pltpu.CompilerParams(use_tc_tiling_on_sc=False),
      out_type=jax.ShapeDtypeStruct((B * CONTEXT, D), jnp.float32),
      scratch_types=[
          pltpu.VMEM((ACT_DMAS, IDX_W), jnp.int32),
          pltpu.VMEM((HALF_ROWS, D), jnp.float32),
          pltpu.SemaphoreType.DMA,
      ],
  )
  def k(act_idx_hbm, act_tab_hbm, act_out, aidx_v, arows_v, sem):
    wid = lax.axis_index("s") * NC + lax.axis_index("c")
    base = wid * BPW
    for j in range(ACT_DMAS):
      pltpu.sync_copy(
          act_idx_hbm.at[pl.ds(base * CONTEXT + j * IDX_W, IDX_W)],
          aidx_v.at[j])
    for h in range(2):
      cps = []
      for j in range(HALF):
        cps.append(pltpu.async_copy(
            act_tab_hbm.at[aidx_v.at[h * HALF + j]],
            arows_v.at[pl.ds(j * IDX_W, IDX_W)], sem))
      for cp in cps:
        cp.wait()
      pltpu.sync_copy(
          arows_v,
          act_out.at[pl.ds(base * CONTEXT + h * HALF_ROWS, HALF_ROWS)])

  return k(act_idx, act_table)


def _sc_trace_tile_gather(trace_idx, tab):
  mesh = plsc.VectorSubcoreMesh(core_axis_name="c", subcore_axis_name="s")

  @functools.partial(
      pl.kernel,
      mesh=mesh,
      out_type=jax.ShapeDtypeStruct((B * 8, D), jnp.float32),
      scratch_types=[
          pltpu.VMEM((BPW,), jnp.int32),
          pltpu.VMEM((BPW * 8 // 2, D), jnp.float32),
          pltpu.SemaphoreType.DMA,
      ],
  )
  def k(idx_hbm, tab_hbm, out_hbm, vidx, stage_v, sem):
    wid = lax.axis_index("s") * NC + lax.axis_index("c")
    base = wid * BPW
    pltpu.sync_copy(idx_hbm.at[pl.ds(base, BPW)], vidx)
    for c in range(2):
      cps = []
      for g in range(BPW // 32):
        vec = vidx[pl.ds(c * (BPW // 2) + g * 16, 16)]
        for j in range(16):
          start = pl.multiple_of((vec[j] >> 3) << 3, 8)
          cps.append(pltpu.async_copy(
              tab_hbm.at[pl.ds(start, 8)],
              stage_v.at[pl.ds((g * 16 + j) * 8, 8)], sem))
      for cp in cps:
        cp.wait()
      pltpu.sync_copy(
          stage_v,
          out_hbm.at[pl.ds((base + c * (BPW // 2)) * 8, BPW * 8 // 2)])

  return k(trace_idx, tab)


BM = 512


def _tc_body(act_ref, st_ref, idx_ref, w1_ref, w2_ref, b_ref, out_ref):
  sub = idx_ref[...] & 7
  emb = st_ref[:, 0, :] * (sub == 0).astype(jnp.float32)
  for s in range(1, 8):
    emb = emb + st_ref[:, s, :] * (sub == s).astype(jnp.float32)
  logits = (
      jnp.dot(act_ref[...], w1_ref[...], preferred_element_type=jnp.float32)
      + jnp.dot(emb, w2_ref[...], preferred_element_type=jnp.float32)
      + b_ref[...]
  )
  m = jnp.max(logits, axis=-1, keepdims=True)
  e = jnp.exp(logits - m)
  out_ref[...] = e / jnp.sum(e, axis=-1, keepdims=True)


def _tc_matmul_softmax(act_flat, staged, trace_i, w1, w2, b2d):
  n = ACT_VOCAB
  ka = CONTEXT * D
  return pl.pallas_call(
      _tc_body,
      grid=(B // BM,),
      in_specs=[
          pl.BlockSpec((BM, ka), lambda i: (i, 0)),
          pl.BlockSpec((BM, 8, D), lambda i: (i, 0, 0)),
          pl.BlockSpec((BM, 1), lambda i: (i, 0)),
          pl.BlockSpec((ka, n), lambda i: (0, 0)),
          pl.BlockSpec((D, n), lambda i: (0, 0)),
          pl.BlockSpec((1, n), lambda i: (0, 0)),
      ],
      out_specs=pl.BlockSpec((BM, n), lambda i: (i, 0)),
      out_shape=jax.ShapeDtypeStruct((B, n), jnp.float32),
  )(act_flat, staged, trace_i, w1, w2, b2d)


def kernel(trace, act_context, act_table, trace_table, W, b):
  act_rows = _sc_act_gather(act_context.reshape(-1), act_table)
  staged2 = _sc_trace_tile_gather(trace.reshape(-1), trace_table)
  staged = staged2.reshape(B, 8, D)
  act_flat = act_rows.reshape(B, CONTEXT * D)
  w1 = W[: CONTEXT * D]
  w2 = W[CONTEXT * D:]
  return _tc_matmul_softmax(act_flat, staged, trace, w1, w2, b.reshape(1, -1))


# speedup vs baseline: 1.0032x; 1.0032x over previous
import functools

import jax
import jax.numpy as jnp
from jax import lax
from jax.experimental import pallas as pl
from jax.experimental.pallas import tpu as pltpu
from jax.experimental.pallas import tpu_sc as plsc

ACT_VOCAB = 1000
D = 64
CONTEXT = 20
B = 4096

NC = 2
NS = 16
NW = NC * NS
BPW = B // NW
IDX_W = 128
ACT_DMAS = BPW * CONTEXT // IDX_W
HALF = ACT_DMAS // 2
HALF_ROWS = HALF * IDX_W


def _sc_act_gather(act_idx, act_table):
  mesh = plsc.VectorSubcoreMesh(core_axis_name="c", subcore_axis_name="s")

  @functools.partial(
      pl.kernel,
      mesh=mesh,
      compiler_params=pltpu.CompilerParams(use_tc_tiling_on_sc=False),
      out_type=jax.ShapeDtypeStruct((B * CONTEXT, D), jnp.float32),
      scratch_types=[
          pltpu.VMEM((ACT_DMAS, IDX_W), jnp.int32),
          pltpu.VMEM((HALF_ROWS, D), jnp.float32),
          pltpu.SemaphoreType.DMA,
      ],
  )
  def k(act_idx_hbm, act_tab_hbm, act_out, aidx_v, arows_v, sem):
    wid = lax.axis_index("s") * NC + lax.axis_index("c")
    base = wid * BPW
    for j in range(ACT_DMAS):
      pltpu.sync_copy(
          act_idx_hbm.at[pl.ds(base * CONTEXT + j * IDX_W, IDX_W)],
          aidx_v.at[j])
    for h in range(2):
      cps = []
      for j in range(HALF):
        cps.append(pltpu.async_copy(
            act_tab_hbm.at[aidx_v.at[h * HALF + j]],
            arows_v.at[pl.ds(j * IDX_W, IDX_W)], sem))
      for cp in cps:
        cp.wait()
      pltpu.sync_copy(
          arows_v,
          act_out.at[pl.ds(base * CONTEXT + h * HALF_ROWS, HALF_ROWS)])

  return k(act_idx, act_table)


def _sc_trace_tile_gather(trace_idx, tab):
  mesh = plsc.VectorSubcoreMesh(core_axis_name="c", subcore_axis_name="s")

  @functools.partial(
      pl.kernel,
      mesh=mesh,
      out_type=jax.ShapeDtypeStruct((B, 8, D), jnp.float32),
      scratch_types=[
          pltpu.VMEM((BPW,), jnp.int32),
          pltpu.VMEM((BPW // 2, 8, D), jnp.float32),
          pltpu.SemaphoreType.DMA,
      ],
  )
  def k(idx_hbm, tab_hbm, out_hbm, vidx, stage_v, sem):
    wid = lax.axis_index("s") * NC + lax.axis_index("c")
    base = wid * BPW
    pltpu.sync_copy(idx_hbm.at[pl.ds(base, BPW)], vidx)
    for c in range(2):
      cps = []
      for g in range(BPW // 32):
        vec = vidx[pl.ds(c * (BPW // 2) + g * 16, 16)]
        for j in range(16):
          start = pl.multiple_of((vec[j] >> 3) << 3, 8)
          cps.append(pltpu.async_copy(
              tab_hbm.at[pl.ds(start, 8)],
              stage_v.at[g * 16 + j], sem))
      for cp in cps:
        cp.wait()
      pltpu.sync_copy(
          stage_v,
          out_hbm.at[pl.ds(base + c * (BPW // 2), BPW // 2)])

  return k(trace_idx, tab)


BM = 512


def _tc_body(act_ref, st_ref, idx_ref, w1_ref, w2_ref, b_ref, out_ref):
  sub = idx_ref[...] & 7
  emb = st_ref[:, 0, :] * (sub == 0).astype(jnp.float32)
  for s in range(1, 8):
    emb = emb + st_ref[:, s, :] * (sub == s).astype(jnp.float32)
  logits = (
      jnp.dot(act_ref[...], w1_ref[...], preferred_element_type=jnp.float32)
      + jnp.dot(emb, w2_ref[...], preferred_element_type=jnp.float32)
      + b_ref[...]
  )
  m = jnp.max(logits, axis=-1, keepdims=True)
  e = jnp.exp(logits - m)
  out_ref[...] = e / jnp.sum(e, axis=-1, keepdims=True)


def _tc_matmul_softmax(act_flat, staged, trace_i, w1, w2, b2d):
  n = ACT_VOCAB
  ka = CONTEXT * D
  return pl.pallas_call(
      _tc_body,
      grid=(B // BM,),
      in_specs=[
          pl.BlockSpec((BM, ka), lambda i: (i, 0)),
          pl.BlockSpec((BM, 8, D), lambda i: (i, 0, 0)),
          pl.BlockSpec((BM, 1), lambda i: (i, 0)),
          pl.BlockSpec((ka, n), lambda i: (0, 0)),
          pl.BlockSpec((D, n), lambda i: (0, 0)),
          pl.BlockSpec((1, n), lambda i: (0, 0)),
      ],
      out_specs=pl.BlockSpec((BM, n), lambda i: (i, 0)),
      out_shape=jax.ShapeDtypeStruct((B, n), jnp.float32),
  )(act_flat, staged, trace_i, w1, w2, b2d)


def kernel(trace, act_context, act_table, trace_table, W, b):
  act_rows = _sc_act_gather(act_context.reshape(-1), act_table)
  staged = _sc_trace_tile_gather(trace.reshape(-1), trace_table)
  act_flat = act_rows.reshape(B, CONTEXT * D)
  w1 = W[: CONTEXT * D]
  w2 = W[CONTEXT * D:]
  return _tc_matmul_softmax(act_flat, staged, trace, w1, w2, b.reshape(1, -1))


# speedup vs baseline: 2.6054x; 2.5970x over previous
import functools

import jax
import jax.numpy as jnp
from jax import lax
from jax.experimental import pallas as pl
from jax.experimental.pallas import tpu as pltpu
from jax.experimental.pallas import tpu_sc as plsc

ACT_VOCAB = 1000
D = 64
CONTEXT = 20
B = 4096

NC = 2
NS = 16
NW = NC * NS
BPW = B // NW
IDX_W = 128
ACT_DMAS = BPW * CONTEXT // IDX_W
HALF = ACT_DMAS // 2
HALF_ROWS = HALF * IDX_W


def _sc_act_gather(act_idx, act_table):
  mesh = plsc.VectorSubcoreMesh(core_axis_name="c", subcore_axis_name="s")

  @functools.partial(
      pl.kernel,
      mesh=mesh,
      compiler_params=pltpu.CompilerParams(use_tc_tiling_on_sc=False),
      out_type=jax.ShapeDtypeStruct((B * CONTEXT, D), jnp.float32),
      scratch_types=[
          pltpu.VMEM((ACT_DMAS, IDX_W), jnp.int32),
          pltpu.VMEM((HALF_ROWS, D), jnp.float32),
          pltpu.SemaphoreType.DMA,
      ],
  )
  def k(act_idx_hbm, act_tab_hbm, act_out, aidx_v, arows_v, sem):
    wid = lax.axis_index("s") * NC + lax.axis_index("c")
    base = wid * BPW
    for j in range(ACT_DMAS):
      pltpu.sync_copy(
          act_idx_hbm.at[pl.ds(base * CONTEXT + j * IDX_W, IDX_W)],
          aidx_v.at[j])
    for h in range(2):
      cps = []
      for j in range(HALF):
        cps.append(pltpu.async_copy(
            act_tab_hbm.at[aidx_v.at[h * HALF + j]],
            arows_v.at[pl.ds(j * IDX_W, IDX_W)], sem))
      for cp in cps:
        cp.wait()
      pltpu.sync_copy(
          arows_v,
          act_out.at[pl.ds(base * CONTEXT + h * HALF_ROWS, HALF_ROWS)])

  return k(act_idx, act_table)


CH = 8


def _sc_trace_gather(trace_idx, tabT):
  mesh = plsc.VectorSubcoreMesh(core_axis_name="c", subcore_axis_name="s")

  @functools.partial(
      pl.kernel,
      mesh=mesh,
      compiler_params=pltpu.CompilerParams(needs_layout_passes=False),
      out_type=jax.ShapeDtypeStruct((B, D), jnp.float32),
      scratch_types=[
          pltpu.VMEM((BPW,), jnp.int32),
          pltpu.VMEM((CH, D, 128), jnp.float32),
          pltpu.VMEM((BPW, D), jnp.float32),
          pltpu.SemaphoreType.DMA,
      ],
  )
  def k(idx_hbm, tab_hbm, out_hbm, vidx, stage_v, rows_v, sem):
    wid = lax.axis_index("s") * NC + lax.axis_index("c")
    base = wid * BPW
    pltpu.sync_copy(idx_hbm.at[pl.ds(base, BPW)], vidx)
    lane = lax.iota(jnp.int32, 16)
    for c in range(BPW // CH):
      rls = []
      cps = []
      for j in range(CH):
        i = c * CH + j
        vec = vidx[pl.ds((i // 16) * 16, 16)]
        r = vec[i % 16]
        start = pl.multiple_of((r >> 7) << 7, 128)
        rls.append(r & 127)
        cps.append(pltpu.async_copy(
            tab_hbm.at[:, pl.ds(start, 128)], stage_v.at[j], sem))
      for cp in cps:
        cp.wait()
      for j in range(CH):
        i = c * CH + j
        j16 = jnp.full((16,), j, jnp.int32)
        l16 = jnp.full((16,), rls[j], jnp.int32)
        for h in range(D // 16):
          v = plsc.load_gather(stage_v, [j16, lane + h * 16, l16])
          rows_v[i, pl.ds(h * 16, 16)] = v
    pltpu.sync_copy(rows_v, out_hbm.at[pl.ds(base, BPW)])

  return k(trace_idx, tabT)


BM = 512


def _tc_body(act_ref, tr_ref, w1_ref, w2_ref, b_ref, out_ref):
  logits = (
      jnp.dot(act_ref[...], w1_ref[...], preferred_element_type=jnp.float32)
      + jnp.dot(tr_ref[...], w2_ref[...], preferred_element_type=jnp.float32)
      + b_ref[...]
  )
  m = jnp.max(logits, axis=-1, keepdims=True)
  e = jnp.exp(logits - m)
  out_ref[...] = e / jnp.sum(e, axis=-1, keepdims=True)


def _tc_matmul_softmax(act_flat, trace_emb, w1, w2, b2d):
  n = ACT_VOCAB
  ka = CONTEXT * D
  return pl.pallas_call(
      _tc_body,
      grid=(B // BM,),
      in_specs=[
          pl.BlockSpec((BM, ka), lambda i: (i, 0)),
          pl.BlockSpec((BM, D), lambda i: (i, 0)),
          pl.BlockSpec((ka, n), lambda i: (0, 0)),
          pl.BlockSpec((D, n), lambda i: (0, 0)),
          pl.BlockSpec((1, n), lambda i: (0, 0)),
      ],
      out_specs=pl.BlockSpec((BM, n), lambda i: (i, 0)),
      out_shape=jax.ShapeDtypeStruct((B, n), jnp.float32),
  )(act_flat, trace_emb, w1, w2, b2d)


def kernel(trace, act_context, act_table, trace_table, W, b):
  act_rows = _sc_act_gather(act_context.reshape(-1), act_table)
  trace_emb = _sc_trace_gather(trace.reshape(-1), trace_table.T)
  act_flat = act_rows.reshape(B, CONTEXT * D)
  w1 = W[: CONTEXT * D]
  w2 = W[CONTEXT * D:]
  return _tc_matmul_softmax(act_flat, trace_emb, w1, w2, b.reshape(1, -1))


# speedup vs baseline: 2.7858x; 1.0692x over previous
import functools

import jax
import jax.numpy as jnp
from jax import lax
from jax.experimental import pallas as pl
from jax.experimental.pallas import tpu as pltpu
from jax.experimental.pallas import tpu_sc as plsc

ACT_VOCAB = 1000
D = 64
CONTEXT = 20
B = 4096

NC = 2
NS = 16
NW = NC * NS
BPW = B // NW
IDX_W = 128
ACT_DMAS = BPW * CONTEXT // IDX_W
HALF = ACT_DMAS // 2
HALF_ROWS = HALF * IDX_W


def _sc_act_gather(act_idx, act_table):
  mesh = plsc.VectorSubcoreMesh(core_axis_name="c", subcore_axis_name="s")

  @functools.partial(
      pl.kernel,
      mesh=mesh,
      compiler_params=pltpu.CompilerParams(use_tc_tiling_on_sc=False),
      out_type=jax.ShapeDtypeStruct((B * CONTEXT, D), jnp.float32),
      scratch_types=[
          pltpu.VMEM((ACT_DMAS, IDX_W), jnp.int32),
          pltpu.VMEM((HALF_ROWS, D), jnp.float32),
          pltpu.SemaphoreType.DMA,
      ],
  )
  def k(act_idx_hbm, act_tab_hbm, act_out, aidx_v, arows_v, sem):
    wid = lax.axis_index("s") * NC + lax.axis_index("c")
    base = wid * BPW
    for j in range(ACT_DMAS):
      pltpu.sync_copy(
          act_idx_hbm.at[pl.ds(base * CONTEXT + j * IDX_W, IDX_W)],
          aidx_v.at[j])
    for h in range(2):
      cps = []
      for j in range(HALF):
        cps.append(pltpu.async_copy(
            act_tab_hbm.at[aidx_v.at[h * HALF + j]],
            arows_v.at[pl.ds(j * IDX_W, IDX_W)], sem))
      for cp in cps:
        cp.wait()
      pltpu.sync_copy(
          arows_v,
          act_out.at[pl.ds(base * CONTEXT + h * HALF_ROWS, HALF_ROWS)])

  return k(act_idx, act_table)


CH = 4


def _sc_trace_gather(trace_idx, tabT):
  mesh = plsc.VectorSubcoreMesh(core_axis_name="c", subcore_axis_name="s")

  @functools.partial(
      pl.kernel,
      mesh=mesh,
      compiler_params=pltpu.CompilerParams(needs_layout_passes=False),
      out_type=jax.ShapeDtypeStruct((B, D), jnp.float32),
      scratch_types=[
          pltpu.VMEM((BPW,), jnp.int32),
          pltpu.VMEM((2, CH, D, 128), jnp.float32),
          pltpu.VMEM((BPW, D), jnp.float32),
          pltpu.SemaphoreType.DMA,
      ],
  )
  def k(idx_hbm, tab_hbm, out_hbm, vidx, stage_v, rows_v, sem):
    wid = lax.axis_index("s") * NC + lax.axis_index("c")
    base = wid * BPW
    pltpu.sync_copy(idx_hbm.at[pl.ds(base, BPW)], vidx)
    lane = lax.iota(jnp.int32, 16)
    nch = BPW // CH
    rls = [None] * nch
    cps = [None] * nch

    def fire(c):
      rl = []
      cp = []
      for j in range(CH):
        i = c * CH + j
        vec = vidx[pl.ds((i // 16) * 16, 16)]
        r = vec[i % 16]
        start = pl.multiple_of((r >> 7) << 7, 128)
        rl.append(r & 127)
        cp.append(pltpu.async_copy(
            tab_hbm.at[:, pl.ds(start, 128)], stage_v.at[c % 2, j], sem))
      rls[c], cps[c] = rl, cp

    fire(0)
    for c in range(nch):
      for cp in cps[c]:
        cp.wait()
      if c + 1 < nch:
        fire(c + 1)
      for j in range(CH):
        i = c * CH + j
        j16 = jnp.full((16,), j, jnp.int32)
        l16 = jnp.full((16,), rls[c][j], jnp.int32)
        c16 = jnp.full((16,), c % 2, jnp.int32)
        for h in range(D // 16):
          v = plsc.load_gather(stage_v, [c16, j16, lane + h * 16, l16])
          rows_v[i, pl.ds(h * 16, 16)] = v
    pltpu.sync_copy(rows_v, out_hbm.at[pl.ds(base, BPW)])

  return k(trace_idx, tabT)


BM = 512


def _tc_body(act_ref, tr_ref, w1_ref, w2_ref, b_ref, out_ref):
  logits = (
      jnp.dot(act_ref[...], w1_ref[...], preferred_element_type=jnp.float32)
      + jnp.dot(tr_ref[...], w2_ref[...], preferred_element_type=jnp.float32)
      + b_ref[...]
  )
  m = jnp.max(logits, axis=-1, keepdims=True)
  e = jnp.exp(logits - m)
  out_ref[...] = (e / jnp.sum(e, axis=-1, keepdims=True)).T


def _tc_matmul_softmax(act_flat, trace_emb, w1, w2, b2d):
  n = ACT_VOCAB
  ka = CONTEXT * D
  outT = pl.pallas_call(
      _tc_body,
      grid=(B // BM,),
      in_specs=[
          pl.BlockSpec((BM, ka), lambda i: (i, 0)),
          pl.BlockSpec((BM, D), lambda i: (i, 0)),
          pl.BlockSpec((ka, n), lambda i: (0, 0)),
          pl.BlockSpec((D, n), lambda i: (0, 0)),
          pl.BlockSpec((1, n), lambda i: (0, 0)),
      ],
      out_specs=pl.BlockSpec((n, BM), lambda i: (0, i)),
      out_shape=jax.ShapeDtypeStruct((n, B), jnp.float32),
  )(act_flat, trace_emb, w1, w2, b2d)
  return outT.T


def kernel(trace, act_context, act_table, trace_table, W, b):
  act_rows = _sc_act_gather(act_context.reshape(-1), act_table)
  trace_emb = _sc_trace_gather(trace.reshape(-1), trace_table.T)
  act_flat = act_rows.reshape(B, CONTEXT * D)
  w1 = W[: CONTEXT * D]
  w2 = W[CONTEXT * D:]
  return _tc_matmul_softmax(act_flat, trace_emb, w1, w2, b.reshape(1, -1))
